# Initial kernel scaffold; baseline (speedup 1.0000x reference)
import jax
import jax.numpy as jnp
from jax.experimental import pallas as pl


def kernel(user_batch, embeddings, sess_items):
    raise NotImplementedError("write your pallas kernel here")


# speedup vs baseline: 5.8444x; 5.8444x over previous
import jax
import jax.numpy as jnp
from jax import lax
from jax.experimental import pallas as pl
from jax.experimental.pallas import tpu as pltpu
from jax.experimental.pallas import tpu_sc as plsc

HIST = 50
EMBED = 64
BATCH = 4096

_info = plsc.get_sparse_core_info()
_NC, _NS, _L = _info.num_cores, _info.num_subcores, _info.num_lanes
_NW = _NC * _NS
_BPW = BATCH // _NW
_VPU = _BPW // _L


def _sc_body(uids_hbm, emb_hbm, sess_flat_hbm, out_hbm,
             uids_v, idx_all, titems, acc_v, sem_i, sem_e):
    wid = lax.axis_index("s") * _NC + lax.axis_index("c")
    base = wid * _BPW

    pltpu.sync_copy(uids_hbm.at[pl.ds(base, _BPW)], uids_v)
    for k in range(_VPU):
        ubase = uids_v[pl.ds(k * _L, _L)] * HIST
        for j in range(HIST):
            idx_all[j, pl.ds(k * _L, _L)] = ubase + j

    hi = []
    for j in range(HIST):
        hi.append(pltpu.async_copy(
            sess_flat_hbm.at[idx_all.at[j]], titems.at[j], sem_i))
    for h in hi:
        h.wait()

    zero = jnp.zeros((_L,), jnp.float32)
    for u in range(_BPW):
        for d in range(EMBED // _L):
            acc_v[u, pl.ds(d * _L, _L)] = zero
    he = []
    for j in range(HIST):
        he.append(pltpu.async_copy(
            emb_hbm.at[titems.at[j]], acc_v, sem_e, add=True))
    for h in he:
        h.wait()

    scale = jnp.float32(1.0 / HIST)
    for u in range(_BPW):
        for d in range(EMBED // _L):
            sl = pl.ds(d * _L, _L)
            acc_v[u, sl] = acc_v[u, sl] * scale
    pltpu.sync_copy(acc_v, out_hbm.at[pl.ds(base, _BPW)])


def kernel(user_batch, embeddings, sess_items):
    uids = user_batch.astype(jnp.int32)
    sess_flat = sess_items.astype(jnp.int32).reshape(-1)
    mesh = plsc.VectorSubcoreMesh(core_axis_name="c", subcore_axis_name="s")
    k = pl.kernel(
        _sc_body,
        mesh=mesh,
        compiler_params=pltpu.CompilerParams(use_tc_tiling_on_sc=False),
        out_type=jax.ShapeDtypeStruct((BATCH, EMBED), jnp.float32),
        scratch_types=[
            pltpu.VMEM((_BPW,), jnp.int32),
            pltpu.VMEM((HIST, _BPW), jnp.int32),
            pltpu.VMEM((HIST, _BPW), jnp.int32),
            pltpu.VMEM((_BPW, EMBED), jnp.float32),
            pltpu.SemaphoreType.DMA,
            pltpu.SemaphoreType.DMA,
        ],
    )
    return k(uids, embeddings, sess_flat)


# speedup vs baseline: 9.2728x; 1.5866x over previous
import jax
import jax.numpy as jnp
from jax import lax
from jax.experimental import pallas as pl
from jax.experimental.pallas import tpu as pltpu
from jax.experimental.pallas import tpu_sc as plsc

HIST = 50
EMBED = 64
BATCH = 4096

_info = plsc.get_sparse_core_info()
_NC, _NS, _L = _info.num_cores, _info.num_subcores, _info.num_lanes
_NW = _NC * _NS
_BPW = BATCH // _NW


def _k1_body(uids_hbm, sess_t_hbm, titems_hbm, uids_v, titems_v, sem_i):
    wid = lax.axis_index("s") * _NC + lax.axis_index("c")
    base = wid * _BPW
    pltpu.sync_copy(uids_hbm.at[pl.ds(base, _BPW)], uids_v)
    hi = []
    for j in range(HIST):
        hi.append(pltpu.async_copy(
            sess_t_hbm.at[j].at[uids_v], titems_v.at[j], sem_i))
    for h in hi:
        h.wait()
    pltpu.sync_copy(titems_v, titems_hbm.at[wid])


def _k2_body(emb_hbm, titems_hbm, out_hbm, titems_v, acc_v, sem_t, sem_e):
    wid = lax.axis_index("s") * _NC + lax.axis_index("c")
    base = wid * _BPW
    pltpu.sync_copy(titems_hbm.at[wid], titems_v)
    zero = jnp.zeros((_L,), jnp.float32)
    for u in range(_BPW):
        for d in range(EMBED // _L):
            acc_v[u, pl.ds(d * _L, _L)] = zero
    he = []
    for j in range(HIST):
        he.append(pltpu.async_copy(
            emb_hbm.at[titems_v.at[j]], acc_v, sem_e, add=True))
    for h in he:
        h.wait()
    scale = jnp.float32(1.0 / HIST)
    for u in range(_BPW):
        for d in range(EMBED // _L):
            sl = pl.ds(d * _L, _L)
            acc_v[u, sl] = acc_v[u, sl] * scale
    pltpu.sync_copy(acc_v, out_hbm.at[pl.ds(base, _BPW)])


def kernel(user_batch, embeddings, sess_items):
    uids = user_batch.astype(jnp.int32)
    sess_t = sess_items.astype(jnp.int32).T
    mesh = plsc.VectorSubcoreMesh(core_axis_name="c", subcore_axis_name="s")
    k1 = pl.kernel(
        _k1_body,
        mesh=mesh,
        compiler_params=pltpu.CompilerParams(use_tc_tiling_on_sc=False),
        out_type=jax.ShapeDtypeStruct((_NW, HIST, _BPW), jnp.int32),
        scratch_types=[
            pltpu.VMEM((_BPW,), jnp.int32),
            pltpu.VMEM((HIST, _BPW), jnp.int32),
            pltpu.SemaphoreType.DMA,
        ],
    )
    titems = k1(uids, sess_t)
    k2 = pl.kernel(
        _k2_body,
        mesh=mesh,
        compiler_params=pltpu.CompilerParams(use_tc_tiling_on_sc=False),
        out_type=jax.ShapeDtypeStruct((BATCH, EMBED), jnp.float32),
        scratch_types=[
            pltpu.VMEM((HIST, _BPW), jnp.int32),
            pltpu.VMEM((_BPW, EMBED), jnp.float32),
            pltpu.SemaphoreType.DMA,
            pltpu.SemaphoreType.DMA,
        ],
    )
    return k2(embeddings, titems)


# speedup vs baseline: 9.2977x; 1.0027x over previous
import jax
import jax.numpy as jnp
from jax import lax
from jax.experimental import pallas as pl
from jax.experimental.pallas import tpu as pltpu
from jax.experimental.pallas import tpu_sc as plsc

HIST = 50
EMBED = 64
BATCH = 4096

_info = plsc.get_sparse_core_info()
_NC, _NS, _L = _info.num_cores, _info.num_subcores, _info.num_lanes
_NW = _NC * _NS
_BPW = BATCH // _NW


def _k1_body(uids_hbm, sess_t_hbm, titems_hbm, uids_v, titems_v, sem_i):
    wid = lax.axis_index("s") * _NC + lax.axis_index("c")
    base = wid * _BPW
    pltpu.sync_copy(uids_hbm.at[pl.ds(base, _BPW)], uids_v)
    hi = []
    for j in range(HIST):
        hi.append(pltpu.async_copy(
            sess_t_hbm.at[j].at[uids_v], titems_v.at[j], sem_i))
    for h in hi:
        h.wait()
    pltpu.sync_copy(titems_v, titems_hbm.at[wid])


def _k2_body(emb_hbm, titems_hbm, out_hbm, titems_v, acc_v, sem_t, sem_e):
    wid = lax.axis_index("s") * _NC + lax.axis_index("c")
    base = wid * _BPW
    tcp = pltpu.async_copy(titems_hbm.at[wid], titems_v, sem_t)
    zero = jnp.zeros((_L,), jnp.float32)
    for u in range(_BPW):
        for d in range(EMBED // _L):
            acc_v[u, pl.ds(d * _L, _L)] = zero
    tcp.wait()
    he = []
    for j in range(HIST):
        he.append(pltpu.async_copy(
            emb_hbm.at[titems_v.at[j]], acc_v, sem_e, add=True))
    for h in he:
        h.wait()
    scale = jnp.float32(1.0 / HIST)
    for u in range(_BPW):
        for d in range(EMBED // _L):
            sl = pl.ds(d * _L, _L)
            acc_v[u, sl] = acc_v[u, sl] * scale
    pltpu.sync_copy(acc_v, out_hbm.at[pl.ds(base, _BPW)])


def kernel(user_batch, embeddings, sess_items):
    uids = user_batch.astype(jnp.int32)
    sess_t = sess_items.astype(jnp.int32).T
    mesh = plsc.VectorSubcoreMesh(core_axis_name="c", subcore_axis_name="s")
    k1 = pl.kernel(
        _k1_body,
        mesh=mesh,
        compiler_params=pltpu.CompilerParams(use_tc_tiling_on_sc=False),
        out_type=jax.ShapeDtypeStruct((_NW, HIST, _BPW), jnp.int32),
        scratch_types=[
            pltpu.VMEM((_BPW,), jnp.int32),
            pltpu.VMEM((HIST, _BPW), jnp.int32),
            pltpu.SemaphoreType.DMA,
        ],
    )
    titems = k1(uids, sess_t)
    k2 = pl.kernel(
        _k2_body,
        mesh=mesh,
        compiler_params=pltpu.CompilerParams(use_tc_tiling_on_sc=False),
        out_type=jax.ShapeDtypeStruct((BATCH, EMBED), jnp.float32),
        scratch_types=[
            pltpu.VMEM((HIST, _BPW), jnp.int32),
            pltpu.VMEM((_BPW, EMBED), jnp.float32),
            pltpu.SemaphoreType.DMA,
            pltpu.SemaphoreType.DMA,
        ],
    )
    return k2(embeddings, titems)
